# baseline (speedup 1.0000x reference)
import jax
import jax.numpy as jnp
from jax.experimental import pallas as pl
from jax.experimental.pallas import tpu as pltpu

B, S, D = 1, 2048, 2048
H, KVH, HD = 16, 8, 128
E, TOPK, FF = 8, 2, 4096
EPS = 1e-6
NEG = -1e30

BT = 256
QKV_COLS = (H + 2 * KVH) * HD
CQKV = 1024
FCH = 1024
NF = FF // FCH

_NT = (((1,), (1,)), ((), ()))


def _qkv_kernel(x_ref, w_ref, rw_ref, out_ref):
    x = x_ref[...]
    ms = jnp.mean(x * x, axis=-1, keepdims=True)
    h = x * jax.lax.rsqrt(ms + EPS) * rw_ref[...]
    out_ref[...] = jax.lax.dot_general(
        h.astype(jnp.bfloat16), w_ref[...], _NT,
        preferred_element_type=jnp.float32)


def _attn_kernel(q_ref, k_ref, v_ref, o_ref):
    qb = pl.program_id(1)
    q = q_ref[...].astype(jnp.bfloat16)
    k = k_ref[...].astype(jnp.bfloat16)
    s = jax.lax.dot_general(q, k, _NT, preferred_element_type=jnp.float32)
    s = s * (HD ** -0.5)
    row = qb * BT + jax.lax.broadcasted_iota(jnp.int32, s.shape, 0)
    col = jax.lax.broadcasted_iota(jnp.int32, s.shape, 1)
    s = jnp.where(col <= row, s, NEG)
    m = jnp.max(s, axis=-1, keepdims=True)
    p = jnp.exp(s - m)
    p = p / jnp.sum(p, axis=-1, keepdims=True)
    o_ref[...] = jnp.dot(p.astype(jnp.bfloat16),
                         v_ref[...].astype(jnp.bfloat16),
                         preferred_element_type=jnp.float32)


def _post_kernel(a_ref, wo_ref, res_ref, rw_ref, wr_ref,
                 x2_ref, h2_ref, cmb_ref):
    a = jax.lax.dot_general(a_ref[...].astype(jnp.bfloat16), wo_ref[...],
                            _NT, preferred_element_type=jnp.float32)
    x2 = a + res_ref[...]
    x2_ref[...] = x2
    ms = jnp.mean(x2 * x2, axis=-1, keepdims=True)
    h2 = x2 * jax.lax.rsqrt(ms + EPS) * rw_ref[...]
    h2b = h2.astype(jnp.bfloat16)
    h2_ref[...] = h2b
    logits = jax.lax.dot_general(h2b, wr_ref[...], _NT,
                                 preferred_element_type=jnp.float32)
    ii = jax.lax.broadcasted_iota(jnp.int32, logits.shape, 1)
    logits = jnp.where(ii < E, logits, NEG)
    m = jnp.max(logits, axis=-1, keepdims=True)
    p = jnp.exp(logits - m)
    p = p / jnp.sum(p, axis=-1, keepdims=True)
    SBLEND = 4e-5
    v1 = jnp.max(p, axis=-1, keepdims=True)
    i1 = jnp.min(jnp.where(p == v1, ii, QKV_COLS), axis=-1, keepdims=True)
    m1 = ii == i1
    p2 = jnp.where(m1, -1.0, p)
    v2 = jnp.max(p2, axis=-1, keepdims=True)
    i2 = jnp.min(jnp.where(p2 == v2, ii, QKV_COLS), axis=-1, keepdims=True)
    m2 = ii == i2
    p3 = jnp.where(m1 | m2, -1.0, p)
    v3 = jnp.max(p3, axis=-1, keepdims=True)
    i3 = jnp.min(jnp.where(p3 == v3, ii, QKV_COLS), axis=-1, keepdims=True)
    m3 = ii == i3
    alpha = 1.0 / (1.0 + jnp.exp((v3 - v2) * (1.0 / SBLEND)))
    cmb_ref[...] = (jnp.where(m1, p, 0.0)
                    + jnp.where(m2, alpha * p, 0.0)
                    + jnp.where(m3, (1.0 - alpha) * p, 0.0))


def _moe_kernel(h2_ref, wg_ref, wu_ref, w2_ref, cmb_ref, out_ref, acc_ref):
    e = pl.program_id(0)
    f = pl.program_id(1)
    r = pl.program_id(2)
    xb = h2_ref[...]
    g = jax.lax.dot_general(xb, wg_ref[0], _NT,
                            preferred_element_type=jnp.float32)
    u = jax.lax.dot_general(xb, wu_ref[0], _NT,
                            preferred_element_type=jnp.float32)
    act = (g * (1.0 / (1.0 + jnp.exp(-g)))) * u
    y = jax.lax.dot_general(act.astype(jnp.bfloat16), w2_ref[0], _NT,
                            preferred_element_type=jnp.float32)
    ii = jax.lax.broadcasted_iota(jnp.int32, cmb_ref.shape, 1)
    c = jnp.sum(jnp.where(ii == e, cmb_ref[...], 0.0), axis=1, keepdims=True)
    contrib = y * c
    rows = pl.ds(r * BT, BT)
    first = (e == 0) & (f == 0)

    @pl.when(first)
    def _():
        acc_ref[rows, :] = contrib

    @pl.when(jnp.logical_not(first))
    def _():
        acc_ref[rows, :] = acc_ref[rows, :] + contrib

    @pl.when((e == E - 1) & (f == NF - 1))
    def _():
        out_ref[...] = acc_ref[rows, :]


def kernel(positions, hidden_states, rms1_w, rms2_w, w_qkv, w_o,
           w_router, ws, w2s):
    f32 = jnp.float32
    x = hidden_states.reshape(S, D)
    rw1 = rms1_w.reshape(1, D)
    rw2 = rms2_w.reshape(1, D)
    wr_pad = jnp.zeros((128, D), f32).at[:E].set(w_router).astype(jnp.bfloat16)
    wqkvb = w_qkv.astype(jnp.bfloat16)
    wob = w_o.astype(jnp.bfloat16)
    wsb = ws.astype(jnp.bfloat16)
    w2sb = w2s.astype(jnp.bfloat16)

    qkv = pl.pallas_call(
        _qkv_kernel,
        grid=(QKV_COLS // CQKV, S // BT),
        in_specs=[
            pl.BlockSpec((BT, D), lambda c, r: (r, 0)),
            pl.BlockSpec((CQKV, D), lambda c, r: (c, 0)),
            pl.BlockSpec((1, D), lambda c, r: (0, 0)),
        ],
        out_specs=pl.BlockSpec((BT, CQKV), lambda c, r: (r, c)),
        out_shape=jax.ShapeDtypeStruct((S, QKV_COLS), f32),
    )(x, wqkvb, rw1)

    attn = pl.pallas_call(
        _attn_kernel,
        grid=(H, S // BT),
        in_specs=[
            pl.BlockSpec((BT, HD), lambda h, qb: (qb, h)),
            pl.BlockSpec((S, HD), lambda h, qb: (0, H + h // 2)),
            pl.BlockSpec((S, HD), lambda h, qb: (0, H + KVH + h // 2)),
        ],
        out_specs=pl.BlockSpec((BT, HD), lambda h, qb: (qb, h)),
        out_shape=jax.ShapeDtypeStruct((S, H * HD), f32),
    )(qkv, qkv, qkv)

    x2, h2b, cmb = pl.pallas_call(
        _post_kernel,
        grid=(S // BT,),
        in_specs=[
            pl.BlockSpec((BT, H * HD), lambda r: (r, 0)),
            pl.BlockSpec((D, H * HD), lambda r: (0, 0)),
            pl.BlockSpec((BT, D), lambda r: (r, 0)),
            pl.BlockSpec((1, D), lambda r: (0, 0)),
            pl.BlockSpec((128, D), lambda r: (0, 0)),
        ],
        out_specs=[
            pl.BlockSpec((BT, D), lambda r: (r, 0)),
            pl.BlockSpec((BT, D), lambda r: (r, 0)),
            pl.BlockSpec((BT, 128), lambda r: (r, 0)),
        ],
        out_shape=[
            jax.ShapeDtypeStruct((S, D), f32),
            jax.ShapeDtypeStruct((S, D), jnp.bfloat16),
            jax.ShapeDtypeStruct((S, 128), f32),
        ],
    )(attn, wob, x, rw2, wr_pad)

    moe = pl.pallas_call(
        _moe_kernel,
        grid=(E, NF, S // BT),
        in_specs=[
            pl.BlockSpec((BT, D), lambda e, f, r: (r, 0)),
            pl.BlockSpec((1, FCH, D), lambda e, f, r: (e, f, 0)),
            pl.BlockSpec((1, FCH, D), lambda e, f, r: (e, NF + f, 0)),
            pl.BlockSpec((1, D, FCH), lambda e, f, r: (e, 0, f)),
            pl.BlockSpec((BT, 128), lambda e, f, r: (r, 0)),
        ],
        out_specs=pl.BlockSpec(
            (BT, D),
            lambda e, f, r: (jnp.where((e == E - 1) & (f == NF - 1), r, 0), 0)),
        out_shape=jax.ShapeDtypeStruct((S, D), f32),
        scratch_shapes=[pltpu.VMEM((S, D), jnp.float32)],
    )(h2b, wsb, wsb, w2sb, cmb)

    return moe.reshape(B, S, D), x2.reshape(B, S, D)


# speedup vs baseline: 1.1030x; 1.1030x over previous
import jax
import jax.numpy as jnp
from jax.experimental import pallas as pl
from jax.experimental.pallas import tpu as pltpu

B, S, D = 1, 2048, 2048
H, KVH, HD = 16, 8, 128
E, TOPK, FF = 8, 2, 4096
EPS = 1e-6
NEG = -1e30

BT = 256
QKV_COLS = (H + 2 * KVH) * HD
CQKV = 1024
FCH = 1024
NF = FF // FCH

_NT = (((1,), (1,)), ((), ()))


def _qkv_kernel(x_ref, w_ref, rw_ref, out_ref):
    x = x_ref[...]
    ms = jnp.mean(x * x, axis=-1, keepdims=True)
    h = x * jax.lax.rsqrt(ms + EPS) * rw_ref[...]
    out_ref[...] = jax.lax.dot_general(
        h.astype(jnp.bfloat16), w_ref[...], _NT,
        preferred_element_type=jnp.float32)


def _attn_kernel(q_ref, k_ref, v_ref, o_ref):
    qb = pl.program_id(1)
    q = q_ref[...].astype(jnp.bfloat16)
    k = k_ref[...].astype(jnp.bfloat16)
    s = jax.lax.dot_general(q, k, _NT, preferred_element_type=jnp.float32)
    s = s * (HD ** -0.5)
    row = qb * BT + jax.lax.broadcasted_iota(jnp.int32, s.shape, 0)
    col = jax.lax.broadcasted_iota(jnp.int32, s.shape, 1)
    s = jnp.where(col <= row, s, NEG)
    m = jnp.max(s, axis=-1, keepdims=True)
    p = jnp.exp(s - m)
    p = p / jnp.sum(p, axis=-1, keepdims=True)
    o_ref[...] = jnp.dot(p.astype(jnp.bfloat16),
                         v_ref[...].astype(jnp.bfloat16),
                         preferred_element_type=jnp.float32)


def _post_kernel(a_ref, wo_ref, res_ref, rw_ref, wr_ref,
                 x2_ref, h2_ref, cmb_ref):
    a = jax.lax.dot_general(a_ref[...].astype(jnp.bfloat16), wo_ref[...],
                            _NT, preferred_element_type=jnp.float32)
    x2 = a + res_ref[...]
    x2_ref[...] = x2
    ms = jnp.mean(x2 * x2, axis=-1, keepdims=True)
    h2 = x2 * jax.lax.rsqrt(ms + EPS) * rw_ref[...]
    h2b = h2.astype(jnp.bfloat16)
    h2_ref[...] = h2b
    logits = jax.lax.dot_general(h2b, wr_ref[...], _NT,
                                 preferred_element_type=jnp.float32)
    ii = jax.lax.broadcasted_iota(jnp.int32, logits.shape, 1)
    logits = jnp.where(ii < E, logits, NEG)
    m = jnp.max(logits, axis=-1, keepdims=True)
    p = jnp.exp(logits - m)
    p = p / jnp.sum(p, axis=-1, keepdims=True)
    SBLEND = 4e-5
    v1 = jnp.max(p, axis=-1, keepdims=True)
    i1 = jnp.min(jnp.where(p == v1, ii, QKV_COLS), axis=-1, keepdims=True)
    m1 = ii == i1
    p2 = jnp.where(m1, -1.0, p)
    v2 = jnp.max(p2, axis=-1, keepdims=True)
    i2 = jnp.min(jnp.where(p2 == v2, ii, QKV_COLS), axis=-1, keepdims=True)
    m2 = ii == i2
    p3 = jnp.where(m1 | m2, -1.0, p)
    v3 = jnp.max(p3, axis=-1, keepdims=True)
    i3 = jnp.min(jnp.where(p3 == v3, ii, QKV_COLS), axis=-1, keepdims=True)
    m3 = ii == i3
    alpha = 1.0 / (1.0 + jnp.exp((v3 - v2) * (1.0 / SBLEND)))
    cmb_ref[...] = (jnp.where(m1, p, 0.0)
                    + jnp.where(m2, alpha * p, 0.0)
                    + jnp.where(m3, (1.0 - alpha) * p, 0.0))


NSLOT = 3
ND = S * NSLOT
BTD = 512
NB = ND // BTD
UNITS = NB + E - 1


def _moeg_kernel(b_ref, ef_ref, em_ref, ff_ref,
                 xs_ref, se_ref, sw_ref, wg_ref, wu_ref, w2_ref, out_ref):
    w = pl.program_id(0)
    f = pl.program_id(1)
    xb = xs_ref[...]
    g = jax.lax.dot_general(xb, wg_ref[0], _NT,
                            preferred_element_type=jnp.float32)
    u = jax.lax.dot_general(xb, wu_ref[0], _NT,
                            preferred_element_type=jnp.float32)
    act = (g * (1.0 / (1.0 + jnp.exp(-g)))) * u
    y = jax.lax.dot_general(act.astype(jnp.bfloat16), w2_ref[0], _NT,
                            preferred_element_type=jnp.float32)
    mask = se_ref[...] == em_ref[w]
    yw = jnp.where(mask, y * sw_ref[...], 0.0)
    first = (ff_ref[w] == 1) & (f == 0)

    @pl.when(first)
    def _():
        out_ref[...] = yw

    @pl.when(jnp.logical_not(first))
    def _():
        out_ref[...] = out_ref[...] + yw


def _moe_kernel(h2_ref, wg_ref, wu_ref, w2_ref, cmb_ref, out_ref, acc_ref):
    e = pl.program_id(0)
    f = pl.program_id(1)
    r = pl.program_id(2)
    xb = h2_ref[...]
    g = jax.lax.dot_general(xb, wg_ref[0], _NT,
                            preferred_element_type=jnp.float32)
    u = jax.lax.dot_general(xb, wu_ref[0], _NT,
                            preferred_element_type=jnp.float32)
    act = (g * (1.0 / (1.0 + jnp.exp(-g)))) * u
    y = jax.lax.dot_general(act.astype(jnp.bfloat16), w2_ref[0], _NT,
                            preferred_element_type=jnp.float32)
    ii = jax.lax.broadcasted_iota(jnp.int32, cmb_ref.shape, 1)
    c = jnp.sum(jnp.where(ii == e, cmb_ref[...], 0.0), axis=1, keepdims=True)
    contrib = y * c
    rows = pl.ds(r * BT, BT)
    first = (e == 0) & (f == 0)

    @pl.when(first)
    def _():
        acc_ref[rows, :] = contrib

    @pl.when(jnp.logical_not(first))
    def _():
        acc_ref[rows, :] = acc_ref[rows, :] + contrib

    @pl.when((e == E - 1) & (f == NF - 1))
    def _():
        out_ref[...] = acc_ref[rows, :]


def kernel(positions, hidden_states, rms1_w, rms2_w, w_qkv, w_o,
           w_router, ws, w2s):
    f32 = jnp.float32
    x = hidden_states.reshape(S, D)
    rw1 = rms1_w.reshape(1, D)
    rw2 = rms2_w.reshape(1, D)
    wr_pad = jnp.zeros((128, D), f32).at[:E].set(w_router).astype(jnp.bfloat16)
    wqkvb = w_qkv.astype(jnp.bfloat16)
    wob = w_o.astype(jnp.bfloat16)
    wsb = ws.astype(jnp.bfloat16)
    w2sb = w2s.astype(jnp.bfloat16)

    qkv = pl.pallas_call(
        _qkv_kernel,
        grid=(QKV_COLS // CQKV, S // BT),
        in_specs=[
            pl.BlockSpec((BT, D), lambda c, r: (r, 0)),
            pl.BlockSpec((CQKV, D), lambda c, r: (c, 0)),
            pl.BlockSpec((1, D), lambda c, r: (0, 0)),
        ],
        out_specs=pl.BlockSpec((BT, CQKV), lambda c, r: (r, c)),
        out_shape=jax.ShapeDtypeStruct((S, QKV_COLS), f32),
    )(x, wqkvb, rw1)

    attn = pl.pallas_call(
        _attn_kernel,
        grid=(H, S // BT),
        in_specs=[
            pl.BlockSpec((BT, HD), lambda h, qb: (qb, h)),
            pl.BlockSpec((S, HD), lambda h, qb: (0, H + h // 2)),
            pl.BlockSpec((S, HD), lambda h, qb: (0, H + KVH + h // 2)),
        ],
        out_specs=pl.BlockSpec((BT, HD), lambda h, qb: (qb, h)),
        out_shape=jax.ShapeDtypeStruct((S, H * HD), f32),
    )(qkv, qkv, qkv)

    x2, h2b, cmb = pl.pallas_call(
        _post_kernel,
        grid=(S // BT,),
        in_specs=[
            pl.BlockSpec((BT, H * HD), lambda r: (r, 0)),
            pl.BlockSpec((D, H * HD), lambda r: (0, 0)),
            pl.BlockSpec((BT, D), lambda r: (r, 0)),
            pl.BlockSpec((1, D), lambda r: (0, 0)),
            pl.BlockSpec((128, D), lambda r: (0, 0)),
        ],
        out_specs=[
            pl.BlockSpec((BT, D), lambda r: (r, 0)),
            pl.BlockSpec((BT, D), lambda r: (r, 0)),
            pl.BlockSpec((BT, 128), lambda r: (r, 0)),
        ],
        out_shape=[
            jax.ShapeDtypeStruct((S, D), f32),
            jax.ShapeDtypeStruct((S, D), jnp.bfloat16),
            jax.ShapeDtypeStruct((S, 128), f32),
        ],
    )(attn, wob, x, rw2, wr_pad)

    i32 = jnp.int32
    vals, idx = jax.lax.top_k(cmb[:, :E], NSLOT)
    eflat = idx.reshape(ND).astype(i32)
    wflat = vals.reshape(ND)
    order = jnp.argsort(eflat).astype(i32)
    se = jnp.take(eflat, order)
    sw = jnp.take(wflat, order)
    st = order // NSLOT
    xs = jnp.take(h2b, st, axis=0)
    seb = se.reshape(NB, BTD)
    e_lo = seb[:, 0]
    e_hi = seb[:, -1]
    nun = e_hi - e_lo + 1
    ustart = jnp.concatenate([jnp.zeros((1,), i32),
                              jnp.cumsum(nun)[:-1].astype(i32)])
    total = ustart[-1] + nun[-1]
    warr = jnp.arange(UNITS, dtype=i32)
    b_of = jnp.clip(jnp.searchsorted(ustart, warr, side='right').astype(i32)
                    - 1, 0, NB - 1)
    e_raw = jnp.take(e_lo, b_of) + (warr - jnp.take(ustart, b_of))
    valid = warr < total
    e_mask = jnp.where(valid, e_raw, E).astype(i32)
    e_fetch = jnp.clip(e_raw, 0, E - 1).astype(i32)
    first_flag = jnp.concatenate(
        [jnp.ones((1,), i32), (b_of[1:] != b_of[:-1]).astype(i32)])

    ydisp = pl.pallas_call(
        _moeg_kernel,
        grid_spec=pltpu.PrefetchScalarGridSpec(
            num_scalar_prefetch=4,
            grid=(UNITS, NF),
            in_specs=[
                pl.BlockSpec((BTD, D), lambda w, f, b, ef, em, ff: (b[w], 0)),
                pl.BlockSpec((BTD, 1), lambda w, f, b, ef, em, ff: (b[w], 0)),
                pl.BlockSpec((BTD, 1), lambda w, f, b, ef, em, ff: (b[w], 0)),
                pl.BlockSpec((1, FCH, D),
                             lambda w, f, b, ef, em, ff: (ef[w], f, 0)),
                pl.BlockSpec((1, FCH, D),
                             lambda w, f, b, ef, em, ff: (ef[w], NF + f, 0)),
                pl.BlockSpec((1, D, FCH),
                             lambda w, f, b, ef, em, ff: (ef[w], 0, f)),
            ],
            out_specs=pl.BlockSpec((BTD, D),
                                   lambda w, f, b, ef, em, ff: (b[w], 0)),
        ),
        out_shape=jax.ShapeDtypeStruct((ND, D), f32),
    )(b_of, e_fetch, e_mask, first_flag,
      xs, se.reshape(ND, 1), sw.reshape(ND, 1), wsb, wsb, w2sb)

    inv = jnp.argsort(order).astype(i32)
    moe = jnp.take(ydisp, inv, axis=0).reshape(S, NSLOT, D).sum(axis=1)

    return moe.reshape(B, S, D), x2.reshape(B, S, D)
